# baseline (speedup 1.0000x reference)
import functools

import jax
import jax.numpy as jnp
from jax import lax
from jax.experimental import pallas as pl
from jax.experimental.pallas import tpu as pltpu
from jax.experimental.pallas import tpu_sc as plsc

_VOCAB = 1000
_B, _L = 1024, 50
_N = _B * _L

_info = plsc.get_sparse_core_info()
_NC = _info.num_cores
_NS = _info.num_subcores
_NW = _NC * _NS
_BPW = _N // _NW
_C = 40
_NCHUNK = _BPW // _C

_mesh = plsc.VectorSubcoreMesh(core_axis_name="c", subcore_axis_name="s")


@functools.partial(
    pl.kernel,
    mesh=_mesh,
    out_type=jax.ShapeDtypeStruct((_N, _VOCAB), jnp.float32),
    scratch_types=[
        pltpu.VMEM((_BPW,), jnp.int32),
        pltpu.VMEM((_C, _VOCAB), jnp.float32),
        pltpu.SemaphoreType.DMA,
    ],
    compiler_params=pltpu.CompilerParams(use_tc_tiling_on_sc=False),
)
def _gather_rows(idx_hbm, table_hbm, out_hbm, idx_v, rows_v, sem):
    wid = lax.axis_index("s") * _NC + lax.axis_index("c")
    base = wid * _BPW
    pltpu.sync_copy(idx_hbm.at[pl.ds(base, _BPW)], idx_v)

    def body(g, carry):
        off = pl.multiple_of(g * _C, 8)
        pltpu.async_copy(table_hbm.at[idx_v.at[pl.ds(off, _C)]], rows_v, sem).wait()
        pltpu.sync_copy(rows_v, out_hbm.at[pl.ds(base + off, _C)])
        return carry

    lax.fori_loop(0, _NCHUNK, body, 0)


def kernel(idx, logits):
    flat = idx.reshape(_N).astype(jnp.int32)
    out = _gather_rows(flat, logits)
    return out.reshape(_B, _L, _VOCAB)


# speedup vs baseline: 1.0365x; 1.0365x over previous
import functools

import jax
import jax.numpy as jnp
from jax import lax
from jax.experimental import pallas as pl
from jax.experimental.pallas import tpu as pltpu
from jax.experimental.pallas import tpu_sc as plsc

_VOCAB = 1000
_B, _L = 1024, 50
_N = _B * _L

_info = plsc.get_sparse_core_info()
_NC = _info.num_cores
_NS = _info.num_subcores
_NW = _NC * _NS
_BPW = _N // _NW
_C = 16
_NCHUNK = _BPW // _C
_DEPTH = 4

_mesh = plsc.VectorSubcoreMesh(core_axis_name="c", subcore_axis_name="s")


@functools.partial(
    pl.kernel,
    mesh=_mesh,
    out_type=jax.ShapeDtypeStruct((_N, _VOCAB), jnp.float32),
    scratch_types=[
        pltpu.VMEM((_BPW,), jnp.int32),
        [pltpu.VMEM((_C, _VOCAB), jnp.float32)] * _DEPTH,
        [pltpu.SemaphoreType.DMA] * _DEPTH,
        [pltpu.SemaphoreType.DMA] * _DEPTH,
    ],
    compiler_params=pltpu.CompilerParams(use_tc_tiling_on_sc=False),
)
def _gather_rows(idx_hbm, table_hbm, out_hbm, idx_v, bufs, semg, semw):
    wid = lax.axis_index("s") * _NC + lax.axis_index("c")
    base = wid * _BPW
    pltpu.sync_copy(idx_hbm.at[pl.ds(base, _BPW)], idx_v)

    def gather(g, buf, sem):
        off = pl.multiple_of(g * _C, 8)
        return pltpu.make_async_copy(
            table_hbm.at[idx_v.at[pl.ds(off, _C)]], buf, sem)

    def write(g, buf, sem):
        off = pl.multiple_of(g * _C, 8)
        return pltpu.make_async_copy(buf, out_hbm.at[pl.ds(base + off, _C)], sem)

    for p in range(_DEPTH - 1):
        gather(p, bufs[p], semg[p]).start()

    def body(g0, carry):
        for p in range(_DEPTH):
            g = g0 + p
            bn = (p + _DEPTH - 1) % _DEPTH
            gather(g, bufs[p], semg[p]).wait()
            write(g, bufs[p], semw[p]).start()

            @pl.when(g >= 1)
            def _():
                write(g - 1, bufs[bn], semw[bn]).wait()

            @pl.when(g + _DEPTH - 1 < _NCHUNK)
            def _():
                gather(g + _DEPTH - 1, bufs[bn], semg[bn]).start()
        return carry

    lax.fori_loop(0, _NCHUNK // _DEPTH, lambda i, c: body(i * _DEPTH, c), 0)
    write(_NCHUNK - 1, bufs[_DEPTH - 1], semw[_DEPTH - 1]).wait()


def kernel(idx, logits):
    flat = idx.reshape(_N).astype(jnp.int32)
    out = _gather_rows(flat, logits)
    return out.reshape(_B, _L, _VOCAB)
